# baseline (speedup 1.0000x reference)
import functools

import jax
import jax.numpy as jnp
from jax import lax
from jax.experimental import pallas as pl
from jax.experimental.pallas import tpu as pltpu
from jax.experimental.pallas import tpu_sc as plsc

DIM = 64
BATCH = 16384

_info = plsc.get_sparse_core_info()
_NC = _info.num_cores
_NS = _info.num_subcores
_NW = _NC * _NS
_BPW = BATCH // _NW
_CH = 128
_NCH = _BPW // _CH

_mesh = plsc.VectorSubcoreMesh(core_axis_name="c", subcore_axis_name="s")


@functools.partial(
    pl.kernel,
    mesh=_mesh,
    out_type=(
        jax.ShapeDtypeStruct((BATCH, DIM), jnp.float32),
        jax.ShapeDtypeStruct((BATCH, DIM), jnp.float32),
    ),
    scratch_types=[
        pltpu.VMEM((_BPW,), jnp.int32),
        pltpu.VMEM((_BPW, DIM), jnp.float32),
        pltpu.VMEM((_BPW, DIM), jnp.float32),
        pltpu.SemaphoreType.DMA,
        pltpu.SemaphoreType.DMA,
    ],
    compiler_params=pltpu.CompilerParams(use_tc_tiling_on_sc=False),
)
def _dual_gather(edge_hbm, node_hbm, feat_hbm, out_e, out_n,
                 idx_v, erows, nrows, sem_e, sem_n):
    wid = lax.axis_index("s") * _NC + lax.axis_index("c")
    base = wid * _BPW
    pltpu.sync_copy(feat_hbm.at[pl.ds(base, _BPW)], idx_v)
    copies = []
    for j in range(_NCH):
        sl = pl.ds(j * _CH, _CH)
        ce = pltpu.async_copy(edge_hbm.at[idx_v.at[sl]], erows.at[sl], sem_e)
        cn = pltpu.async_copy(node_hbm.at[idx_v.at[sl]], nrows.at[sl], sem_n)
        copies.append((ce, cn))
    for j, (ce, cn) in enumerate(copies):
        sl = pl.ds(j * _CH, _CH)
        out_sl = pl.ds(base + j * _CH, _CH)
        ce.wait()
        pltpu.sync_copy(erows.at[sl], out_e.at[out_sl])
        cn.wait()
        pltpu.sync_copy(nrows.at[sl], out_n.at[out_sl])


def kernel(edge_emb, node_emb, feat):
    return _dual_gather(edge_emb, node_emb, feat)
